# Initial kernel scaffold; baseline (speedup 1.0000x reference)
import jax
import jax.numpy as jnp
from jax.experimental import pallas as pl


def kernel(curr_emb, msg, e_count, W1, b1, W2, b2, e_type):
    raise NotImplementedError("write your pallas kernel here")


# speedup vs baseline: 45.8592x; 45.8592x over previous
import jax
import jax.numpy as jnp
from jax.experimental import pallas as pl
from jax.experimental.pallas import tpu as pltpu

_N, _D, _EMB, _K = 10000, 32, 128, 16
_BN = 400


def _tc_body(ec0_ref, et_ref, msg_ref, w1t_ref, w2t_ref, b_ref, out_ref):
    ec0 = ec0_ref[...]
    et = et_ref[...]
    e_total = jnp.sum(ec0, axis=1, keepdims=True)
    gathered = jnp.zeros(et.shape, jnp.float32)
    for k in range(_K):
        gathered = gathered + jnp.where(et == k, ec0[:, k:k + 1], 0.0)
    w = 1.0 / gathered
    msg = msg_ref[...]
    nei = jnp.sum(msg * w[:, :, None], axis=1)
    norm = jnp.sum(msg, axis=1) / e_total
    out1 = jnp.dot(nei, w1t_ref[...], preferred_element_type=jnp.float32)
    out2 = jnp.dot(norm, w2t_ref[...], preferred_element_type=jnp.float32)
    out_ref[...] = jnp.concatenate([out1, out2], axis=1) + b_ref[...]


def kernel(curr_emb, msg, e_count, W1, b1, W2, b2, e_type):
    del curr_emb
    ec0 = e_count[:, 0, :]
    w1t = W1.T
    w2t = W2.T
    b = jnp.concatenate([b1, b2])[None, :]
    grid = (_N // _BN,)
    out = pl.pallas_call(
        _tc_body,
        grid=grid,
        in_specs=[
            pl.BlockSpec((_BN, _K), lambda i: (i, 0)),
            pl.BlockSpec((_BN, _D), lambda i: (i, 0)),
            pl.BlockSpec((_BN, _D, _EMB), lambda i: (i, 0, 0)),
            pl.BlockSpec((_EMB, _EMB // 2), lambda i: (0, 0)),
            pl.BlockSpec((_EMB, _EMB // 2), lambda i: (0, 0)),
            pl.BlockSpec((1, _EMB), lambda i: (0, 0)),
        ],
        out_specs=pl.BlockSpec((_BN, _EMB), lambda i: (i, 0)),
        out_shape=jax.ShapeDtypeStruct((_N, _EMB), jnp.float32),
    )(ec0, e_type, msg, w1t, w2t, b)
    return out
